# baseline (speedup 1.0000x reference)
import jax
import jax.numpy as jnp
from jax.experimental import pallas as pl


def _tc_body(x_ref, w_ref, o_ref):
    bblk, C = x_ref.shape[0], x_ref.shape[1]
    we = w_ref[...]
    for b in range(bblk):
        xm = x_ref[b].reshape(C * 8, 128)
        m = jax.lax.dot_general(
            we, xm,
            dimension_numbers=(((1,), (0,)), ((), ())),
            preferred_element_type=jnp.float32,
        )
        o_ref[b, :] = jnp.sum(m, axis=1) * (1.0 / 1024.0)


def kernel(x, W):
    B, C, H, Wsp = x.shape
    S = H * Wsp
    E = W.shape[0]
    xr = x.reshape(B, C, S // 128, 128)
    Wexp = jnp.repeat(W, S // 128, axis=1)
    Bblk = 8
    return pl.pallas_call(
        _tc_body,
        grid=(B // Bblk,),
        in_specs=[
            pl.BlockSpec((Bblk, C, S // 128, 128), lambda i: (i, 0, 0, 0)),
            pl.BlockSpec((E, C * (S // 128)), lambda i: (0, 0)),
        ],
        out_specs=pl.BlockSpec((Bblk, E), lambda i: (i, 0)),
        out_shape=jax.ShapeDtypeStruct((B, E), jnp.float32),
    )(xr, Wexp)


# speedup vs baseline: 3.8087x; 3.8087x over previous
import jax
import jax.numpy as jnp
from jax.experimental import pallas as pl


def _tc_body(x_ref, w_ref, o_ref):
    inv = 1.0 / x_ref.shape[1]
    s = jnp.sum(x_ref[...], axis=1)
    o_ref[...] = jax.lax.dot_general(
        s, w_ref[...],
        dimension_numbers=(((1,), (1,)), ((), ())),
        preferred_element_type=jnp.float32,
    ) * inv


def kernel(x, W):
    B, C, H, Wsp = x.shape
    S = H * Wsp
    E = W.shape[0]
    xv = jnp.transpose(x, (0, 2, 3, 1)).reshape(B, S, C)
    Bblk = 8
    return pl.pallas_call(
        _tc_body,
        grid=(B // Bblk,),
        in_specs=[
            pl.BlockSpec((Bblk, S, C), lambda i: (i, 0, 0)),
            pl.BlockSpec((E, C), lambda i: (0, 0)),
        ],
        out_specs=pl.BlockSpec((Bblk, E), lambda i: (i, 0)),
        out_shape=jax.ShapeDtypeStruct((B, E), jnp.float32),
    )(xv, W)
